# baseline (speedup 1.0000x reference)
import functools

import jax
import jax.numpy as jnp
from jax import lax
from jax.experimental import pallas as pl
from jax.experimental.pallas import tpu as pltpu
from jax.experimental.pallas import tpu_sc as plsc

_SCALE = 11.313708498984761

_VOCAB = 100000
_DIM = 128
_B = 4096 * 200
_NC = 2
_NS = 16
_NW = _NC * _NS
_PW = _B // _NW
_CHUNK = 64
_CPW = _PW // _CHUNK
_NBUF = 10


def _scale_body(w_ref, o_ref):
    o_ref[...] = w_ref[...] * _SCALE


_SCALE_BLOCK = 20000


@jax.jit
def _scale_table(w):
    return pl.pallas_call(
        _scale_body,
        grid=(_VOCAB // _SCALE_BLOCK,),
        in_specs=[pl.BlockSpec((_SCALE_BLOCK, _DIM), lambda i: (i, 0))],
        out_specs=pl.BlockSpec((_SCALE_BLOCK, _DIM), lambda i: (i, 0)),
        out_shape=jax.ShapeDtypeStruct((_VOCAB, _DIM), jnp.float32),
    )(w)


def _gather_body(w_hbm, x_hbm, out_hbm, idx_v, *rest):
    bufs = rest[:_NBUF]
    gsems = rest[_NBUF:2 * _NBUF]
    osems = rest[2 * _NBUF:]
    wid = lax.axis_index("s") * _NC + lax.axis_index("c")
    obase = wid * _PW

    pltpu.sync_copy(x_hbm.at[pl.ds(obase, _PW)], idx_v)

    for b in range(_NBUF):
        pltpu.async_copy(
            w_hbm.at[idx_v.at[pl.ds(b * _CHUNK, _CHUNK)]], bufs[b], gsems[b])

    def step(i, carry):
        for b in range(_NBUF):
            g = i * _NBUF + b
            pltpu.make_async_copy(
                w_hbm.at[idx_v.at[pl.ds(g * _CHUNK, _CHUNK)]],
                bufs[b], gsems[b]).wait()
            pltpu.async_copy(
                bufs[b], out_hbm.at[pl.ds(obase + g * _CHUNK, _CHUNK)], osems[b])
            nxt = g + _NBUF

            @pl.when(nxt < _CPW)
            def _():
                pltpu.make_async_copy(
                    bufs[b], out_hbm.at[pl.ds(obase, _CHUNK)], osems[b]).wait()
                pltpu.async_copy(
                    w_hbm.at[idx_v.at[pl.ds(nxt * _CHUNK, _CHUNK)]],
                    bufs[b], gsems[b])
        return carry

    lax.fori_loop(0, _CPW // _NBUF, step, 0)

    for b in range(_NBUF):
        pltpu.make_async_copy(
            bufs[b], out_hbm.at[pl.ds(obase, _CHUNK)], osems[b]).wait()


@jax.jit
def _gather(w_scaled, x2d):
    mesh = plsc.VectorSubcoreMesh(core_axis_name="c", subcore_axis_name="s")
    run = pl.kernel(
        _gather_body,
        mesh=mesh,
        out_type=jax.ShapeDtypeStruct((_B, _DIM), jnp.float32),
        scratch_types=(
            [pltpu.VMEM((_PW,), jnp.int32)]
            + [pltpu.VMEM((_CHUNK, _DIM), jnp.float32)] * _NBUF
            + [pltpu.SemaphoreType.DMA] * (2 * _NBUF)
        ),
    )
    return run(w_scaled, x2d)


def kernel(x, W):
    x1d = x.reshape(-1).astype(jnp.int32)
    w_scaled = _scale_table(W)
    out = _gather(w_scaled, x1d)
    dummy = _scale_table(W + 1.0)
    patch = out[0:1, 0:1] + 0.0 * dummy[0:1, 0:1]
    out = jax.lax.dynamic_update_slice(out, patch, (0, 0))
    return out.reshape(x.shape[0], x.shape[1], _DIM)


# speedup vs baseline: 1.1836x; 1.1836x over previous
import functools

import jax
import jax.numpy as jnp
from jax import lax
from jax.experimental import pallas as pl
from jax.experimental.pallas import tpu as pltpu
from jax.experimental.pallas import tpu_sc as plsc

_SCALE = 11.313708498984761

_VOCAB = 100000
_DIM = 128
_B = 4096 * 200
_NC = 2
_NS = 16
_NW = _NC * _NS
_PW = _B // _NW
_CHUNK = 64
_CPW = _PW // _CHUNK
_NBUF = 10


def _scale_body(w_ref, o_ref):
    o_ref[...] = w_ref[...] * _SCALE


_SCALE_BLOCK = 20000


@jax.jit
def _scale_table(w):
    return pl.pallas_call(
        _scale_body,
        grid=(_VOCAB // _SCALE_BLOCK,),
        in_specs=[pl.BlockSpec((_SCALE_BLOCK, _DIM), lambda i: (i, 0))],
        out_specs=pl.BlockSpec((_SCALE_BLOCK, _DIM), lambda i: (i, 0)),
        out_shape=jax.ShapeDtypeStruct((_VOCAB, _DIM), jnp.float32),
    )(w)


def _gather_body(w_hbm, x_hbm, out_hbm, idx_v, *rest):
    bufs = rest[:_NBUF]
    gsems = rest[_NBUF:2 * _NBUF]
    osems = rest[2 * _NBUF:]
    wid = lax.axis_index("s") * _NC + lax.axis_index("c")
    obase = wid * _PW

    pltpu.sync_copy(x_hbm.at[pl.ds(obase, _PW)], idx_v)

    for b in range(_NBUF):
        pltpu.async_copy(
            w_hbm.at[idx_v.at[pl.ds(b * _CHUNK, _CHUNK)]], bufs[b], gsems[b])

    def step(i, carry):
        for b in range(_NBUF):
            g = i * _NBUF + b
            pltpu.make_async_copy(
                w_hbm.at[idx_v.at[pl.ds(g * _CHUNK, _CHUNK)]],
                bufs[b], gsems[b]).wait()
            pltpu.async_copy(
                bufs[b], out_hbm.at[pl.ds(obase + g * _CHUNK, _CHUNK)], osems[b])
            nxt = g + _NBUF

            @pl.when(nxt < _CPW)
            def _():
                pltpu.make_async_copy(
                    bufs[b], out_hbm.at[pl.ds(obase, _CHUNK)], osems[b]).wait()
                pltpu.async_copy(
                    w_hbm.at[idx_v.at[pl.ds(nxt * _CHUNK, _CHUNK)]],
                    bufs[b], gsems[b])
        return carry

    lax.fori_loop(0, _CPW // _NBUF, step, 0)

    for b in range(_NBUF):
        pltpu.make_async_copy(
            bufs[b], out_hbm.at[pl.ds(obase, _CHUNK)], osems[b]).wait()


@jax.jit
def _gather(w_scaled, x2d):
    mesh = plsc.VectorSubcoreMesh(core_axis_name="c", subcore_axis_name="s")
    run = pl.kernel(
        _gather_body,
        mesh=mesh,
        out_type=jax.ShapeDtypeStruct((_B, _DIM), jnp.float32),
        scratch_types=(
            [pltpu.VMEM((_PW,), jnp.int32)]
            + [pltpu.VMEM((_CHUNK, _DIM), jnp.float32)] * _NBUF
            + [pltpu.SemaphoreType.DMA] * (2 * _NBUF)
        ),
    )
    return run(w_scaled, x2d)


def kernel(x, W):
    x1d = x.reshape(-1).astype(jnp.int32)
    w_scaled = _scale_table(W)
    out = _gather(w_scaled, x1d)
    return out.reshape(x.shape[0], x.shape[1], _DIM)


# speedup vs baseline: 1.1846x; 1.0009x over previous
import functools

import jax
import jax.numpy as jnp
from jax import lax
from jax.experimental import pallas as pl
from jax.experimental.pallas import tpu as pltpu
from jax.experimental.pallas import tpu_sc as plsc

_SCALE = 11.313708498984761

_VOCAB = 100000
_DIM = 128
_B = 4096 * 200
_NC = 2
_NS = 16
_NW = _NC * _NS
_PW = _B // _NW
_CHUNK = 64
_CPW = _PW // _CHUNK
_NBUF = 10


def _scale_body(w_ref, o_ref):
    o_ref[...] = w_ref[...] * _SCALE


_SCALE_BLOCK = 25000


@jax.jit
def _scale_table(w):
    return pl.pallas_call(
        _scale_body,
        grid=(_VOCAB // _SCALE_BLOCK,),
        in_specs=[pl.BlockSpec((_SCALE_BLOCK, _DIM), lambda i: (i, 0))],
        out_specs=pl.BlockSpec((_SCALE_BLOCK, _DIM), lambda i: (i, 0)),
        out_shape=jax.ShapeDtypeStruct((_VOCAB, _DIM), jnp.float32),
    )(w)


def _gather_body(w_hbm, x_hbm, out_hbm, idx_v, *rest):
    bufs = rest[:_NBUF]
    gsems = rest[_NBUF:2 * _NBUF]
    osems = rest[2 * _NBUF:]
    wid = lax.axis_index("s") * _NC + lax.axis_index("c")
    obase = wid * _PW

    pltpu.sync_copy(x_hbm.at[pl.ds(obase, _PW)], idx_v)

    for b in range(_NBUF):
        pltpu.async_copy(
            w_hbm.at[idx_v.at[pl.ds(b * _CHUNK, _CHUNK)]], bufs[b], gsems[b])

    def step(i, carry):
        for b in range(_NBUF):
            g = i * _NBUF + b
            pltpu.make_async_copy(
                w_hbm.at[idx_v.at[pl.ds(g * _CHUNK, _CHUNK)]],
                bufs[b], gsems[b]).wait()
            pltpu.async_copy(
                bufs[b], out_hbm.at[pl.ds(obase + g * _CHUNK, _CHUNK)], osems[b])
            nxt = g + _NBUF

            @pl.when(nxt < _CPW)
            def _():
                pltpu.make_async_copy(
                    bufs[b], out_hbm.at[pl.ds(obase, _CHUNK)], osems[b]).wait()
                pltpu.async_copy(
                    w_hbm.at[idx_v.at[pl.ds(nxt * _CHUNK, _CHUNK)]],
                    bufs[b], gsems[b])
        return carry

    lax.fori_loop(0, _CPW // _NBUF, step, 0)

    for b in range(_NBUF):
        pltpu.make_async_copy(
            bufs[b], out_hbm.at[pl.ds(obase, _CHUNK)], osems[b]).wait()


@jax.jit
def _gather(w_scaled, x2d):
    mesh = plsc.VectorSubcoreMesh(core_axis_name="c", subcore_axis_name="s")
    run = pl.kernel(
        _gather_body,
        mesh=mesh,
        out_type=jax.ShapeDtypeStruct((_B, _DIM), jnp.float32),
        scratch_types=(
            [pltpu.VMEM((_PW,), jnp.int32)]
            + [pltpu.VMEM((_CHUNK, _DIM), jnp.float32)] * _NBUF
            + [pltpu.SemaphoreType.DMA] * (2 * _NBUF)
        ),
    )
    return run(w_scaled, x2d)


def kernel(x, W):
    x1d = x.reshape(-1).astype(jnp.int32)
    w_scaled = _scale_table(W)
    out = _gather(w_scaled, x1d)
    return out.reshape(x.shape[0], x.shape[1], _DIM)
